# Initial kernel scaffold; baseline (speedup 1.0000x reference)
import jax
import jax.numpy as jnp
from jax.experimental import pallas as pl


def kernel(data, C0, C1, C2, E0, E1, E2):
    raise NotImplementedError("write your pallas kernel here")


# speedup vs baseline: 1.2147x; 1.2147x over previous
import functools

import jax
import jax.numpy as jnp
from jax import lax
from jax.experimental import pallas as pl
from jax.experimental.pallas import tpu as pltpu
from jax.experimental.pallas import tpu_sc as plsc

P = 16
D = 768
N = 4096
B, C, H, W = 4, 3, 512, 512
HP, WP = H // P, W // P

_NC, _NS = 2, 16
_NW = _NC * _NS
_BPW = N // _NW
_GCH = 64


def _patchify(x):
    b, c, Hh, Ww = x.shape
    h, w = Hh // P, Ww // P
    x = x.reshape(b, c, h, P, w, P)
    x = jnp.transpose(x, (0, 2, 4, 1, 3, 5))
    return x.reshape(b * h * w, c * P * P)


def _unpatchify(x):
    x = x.reshape(B, HP, WP, C, P, P)
    x = jnp.transpose(x, (0, 3, 1, 4, 2, 5))
    return x.reshape(B, C, H, W)


def _stage_body(n_sub, nk, bk, x_ref, *rest):
    sub_refs = rest[:n_sub]
    c_ref = rest[n_sub]
    cls_ref = rest[n_sub + 1]
    bv_ref = rest[n_sub + 2]
    bi_ref = rest[n_sub + 3]
    k = pl.program_id(1)

    rows = x_ref[...]
    for s in sub_refs:
        rows = rows - s[...]
    c = c_ref[...]
    cn = jnp.sum(c * c, axis=1, keepdims=True)
    cb = c.astype(jnp.bfloat16)
    rb = rows.astype(jnp.bfloat16)
    m = lax.dot_general(cb, rb, (((1,), (1,)), ((), ())),
                        preferred_element_type=jnp.float32)
    r2 = (rows * rows).astype(jnp.bfloat16)
    xx = lax.dot_general(jnp.ones((1, D), jnp.bfloat16), r2,
                         (((1,), (1,)), ((), ())),
                         preferred_element_type=jnp.float32)
    d2 = (xx - 2.0 * m) + cn
    scores = jnp.sqrt(jnp.maximum(d2, 0.0))
    bm = jnp.min(scores, axis=0, keepdims=True)
    iota = lax.broadcasted_iota(jnp.int32, scores.shape, 0)
    bidx = jnp.min(jnp.where(scores == bm, iota, jnp.int32(2**30)),
                   axis=0, keepdims=True) + k * bk

    @pl.when(k == 0)
    def _():
        bv_ref[...] = bm
        bi_ref[...] = bidx

    @pl.when(k > 0)
    def _():
        upd = bm < bv_ref[...]
        bv_ref[...] = jnp.where(upd, bm, bv_ref[...])
        bi_ref[...] = jnp.where(upd, bidx, bi_ref[...])

    @pl.when(k == nk - 1)
    def _():
        cls_ref[...] = bi_ref[0]


def _argmin_stage(x, subs, cents, bn, bk):
    K = cents.shape[0]
    nk = K // bk
    n_sub = len(subs)
    row_spec = pl.BlockSpec((bn, D), lambda i, k: (i, 0))
    return pl.pallas_call(
        functools.partial(_stage_body, n_sub, nk, bk),
        grid=(N // bn, nk),
        in_specs=[row_spec] * (1 + n_sub)
        + [pl.BlockSpec((bk, D), lambda i, k: (k, 0))],
        out_specs=pl.BlockSpec((bn,), lambda i, k: (i,)),
        out_shape=jax.ShapeDtypeStruct((N,), jnp.int32),
        scratch_shapes=[
            pltpu.VMEM((1, bn), jnp.float32),
            pltpu.VMEM((1, bn), jnp.int32),
        ],
        compiler_params=pltpu.CompilerParams(
            dimension_semantics=("parallel", "arbitrary")),
    )(x, *subs, cents)


def _sc_gather_pair(cents, emb, cls):
    mesh = plsc.VectorSubcoreMesh(core_axis_name="c", subcore_axis_name="s")

    @functools.partial(
        pl.kernel,
        out_type=(
            jax.ShapeDtypeStruct((N, D), jnp.float32),
            jax.ShapeDtypeStruct((N, D), jnp.float32),
        ),
        mesh=mesh,
        scratch_types=[
            pltpu.VMEM((_GCH,), jnp.int32),
            pltpu.VMEM((_GCH, D), jnp.float32),
            pltpu.VMEM((_GCH, D), jnp.float32),
            pltpu.SemaphoreType.DMA,
        ],
    )
    def k(c_hbm, e_hbm, cls_hbm, g_hbm, eo_hbm, idx_v, bufc, bufe, sem):
        wid = lax.axis_index("s") * _NC + lax.axis_index("c")
        base = wid * _BPW
        for ch in range(_BPW // _GCH):
            off = base + ch * _GCH
            pltpu.sync_copy(cls_hbm.at[pl.ds(off, _GCH)], idx_v)
            cp1 = pltpu.async_copy(c_hbm.at[idx_v], bufc, sem)
            cp2 = pltpu.async_copy(e_hbm.at[idx_v], bufe, sem)
            cp1.wait()
            cp2.wait()
            pltpu.sync_copy(bufc, g_hbm.at[pl.ds(off, _GCH)])
            pltpu.sync_copy(bufe, eo_hbm.at[pl.ds(off, _GCH)])

    return k(cents, emb, cls)


def _final_body(x_ref, g0_ref, g1_ref, g2_ref, e0_ref, e1_ref, e2_ref,
                diff_ref, img_ref, emb_ref):
    g = g0_ref[...] + g1_ref[...] + g2_ref[...]
    img_ref[...] = g
    diff_ref[...] = x_ref[...] - g
    emb_ref[...] = e0_ref[...] + e1_ref[...] + e2_ref[...]


def _final_combine(x, g0, g1, g2, e0, e1, e2):
    bn = 512
    spec = pl.BlockSpec((bn, D), lambda i: (i, 0))
    return pl.pallas_call(
        _final_body,
        grid=(N // bn,),
        in_specs=[spec] * 7,
        out_specs=[spec] * 3,
        out_shape=[jax.ShapeDtypeStruct((N, D), jnp.float32)] * 3,
        compiler_params=pltpu.CompilerParams(
            dimension_semantics=("parallel",)),
    )(x, g0, g1, g2, e0, e1, e2)


def kernel(data, C0, C1, C2, E0, E1, E2):
    x = _patchify(data)

    x = lax.optimization_barrier(x)
    cls0 = _argmin_stage(x, [], C0, bn=1024, bk=512)
    g0, e0 = lax.optimization_barrier(_sc_gather_pair(C0, E0, cls0))

    cls1 = _argmin_stage(x, [g0], C1, bn=1024, bk=512)
    g1, e1 = lax.optimization_barrier(_sc_gather_pair(C1, E1, cls1))

    cls2 = _argmin_stage(x, [g0, g1], C2, bn=1024, bk=512)
    g2, e2 = lax.optimization_barrier(_sc_gather_pair(C2, E2, cls2))

    diff, img, emb = lax.optimization_barrier(
        _final_combine(x, g0, g1, g2, e0, e1, e2))

    patch_embed = jnp.transpose(emb.reshape(B, HP, WP, D), (0, 3, 1, 2))
    img_sum = _unpatchify(img)
    patch_diff = jnp.transpose(diff.reshape(B, HP, WP, D), (0, 3, 1, 2))
    return (patch_embed, img_sum, patch_diff)
